# baseline (speedup 1.0000x reference)
import dataclasses
import functools

import jax
import jax.numpy as jnp
from jax import lax
from jax.experimental import pallas as pl
from jax.experimental.pallas import tpu as pltpu
from jax.experimental.pallas import tpu_sc as plsc

B = 1024
M = 1024
P = 2
DIM = 3
OUT_DIM = 2048

NUM_CORES = 2
NUM_SUBCORES = 16
LANES = 16
NUM_TILES = NUM_CORES * NUM_SUBCORES
CURVES_PER_TILE = B // NUM_TILES
NUM_CHUNKS = OUT_DIM // LANES
CPW = M * 4
OBW = OUT_DIM * DIM


def _sc_curve_eval(inp_flat, nut, ibase4):
  mesh = plsc.VectorSubcoreMesh(core_axis_name="c", subcore_axis_name="s")
  cp = pltpu.CompilerParams()
  if "needs_layout_passes" in pltpu.CompilerParams.__dataclass_fields__:
    cp = dataclasses.replace(cp, needs_layout_passes=False)

  G = 4
  ngroups = CURVES_PER_TILE // G
  npairs = ngroups // 2

  @functools.partial(
      pl.kernel,
      compiler_params=cp,
      out_type=jax.ShapeDtypeStruct((B * OUT_DIM * DIM,), jnp.float32),
      mesh=mesh,
      scratch_types=(
          [pltpu.VMEM((CPW,), jnp.float32)] * (2 * G)
          + [pltpu.VMEM((OBW,), jnp.float32)] * (2 * G)
          + [
              pltpu.VMEM((P + 1, OUT_DIM), jnp.float32),
              pltpu.VMEM((OUT_DIM,), jnp.int32),
              pltpu.SemaphoreType.DMA,
              pltpu.SemaphoreType.DMA,
              pltpu.SemaphoreType.DMA,
              pltpu.SemaphoreType.DMA,
              pltpu.SemaphoreType.DMA,
          ]
      ),
  )
  def k(inp_hbm, nut_hbm, idx_hbm, out_hbm, *scratch):
    cpbuf = [scratch[:G], scratch[G:2 * G]]
    obuf = [scratch[2 * G:3 * G], scratch[3 * G:4 * G]]
    nut_v, idx_v, isem0, isem1, osem0, osem1, csem = scratch[4 * G:]
    isem = [isem0, isem1]
    osem = [osem0, osem1]
    wid = lax.axis_index("s") * NUM_CORES + lax.axis_index("c")
    base = wid * CURVES_PER_TILE

    def start_in(buf, b0):
      for g in range(G):
        pltpu.async_copy(inp_hbm.at[pl.ds((b0 + g) * CPW, CPW)],
                         cpbuf[buf][g], isem[buf])

    def wait_in(buf, b0):
      for g in range(G):
        pltpu.make_async_copy(inp_hbm.at[pl.ds((b0 + g) * CPW, CPW)],
                              cpbuf[buf][g], isem[buf]).wait()

    def start_out(buf, b0):
      for g in range(G):
        pltpu.async_copy(obuf[buf][g],
                         out_hbm.at[pl.ds((b0 + g) * OBW, OBW)], osem[buf])

    def wait_out(buf, b0):
      for g in range(G):
        pltpu.make_async_copy(obuf[buf][g],
                              out_hbm.at[pl.ds((b0 + g) * OBW, OBW)],
                              osem[buf]).wait()

    pltpu.async_copy(nut_hbm, nut_v, csem)
    pltpu.async_copy(idx_hbm, idx_v, csem)
    start_in(0, base)
    pltpu.make_async_copy(nut_hbm, nut_v, csem).wait()
    pltpu.make_async_copy(idx_hbm, idx_v, csem).wait()
    lane3 = lax.iota(jnp.int32, LANES) * 3

    def compute(buf):
      @pl.loop(0, NUM_CHUNKS)
      def _(kk):
        t0 = kk * LANES
        ibase = idx_v[pl.ds(t0, LANES)]
        nu = [nut_v[j, pl.ds(t0, LANES)] for j in range(3)]
        gidx = [ibase + n for n in range(12)]
        obase = lane3 + t0 * 3
        for g in range(G):
          cpg = cpbuf[buf][g]
          acc = [None] * 4
          for c in range(4):
            gs = [plsc.load_gather(cpg, [gidx[4 * j + c]]) for j in range(3)]
            acc[c] = nu[0] * gs[0] + nu[1] * gs[1] + nu[2] * gs[2]
          winv = 1.0 / acc[3]
          og = obuf[buf][g]
          plsc.store_scatter(og, [obase], acc[0] * winv)
          plsc.store_scatter(og, [obase + 1], acc[1] * winv)
          plsc.store_scatter(og, [obase + 2], acc[2] * winv)

    @pl.loop(0, npairs)
    def _(ii):
      b0 = base + 2 * G * ii
      b1 = b0 + G
      wait_in(0, b0)
      start_in(1, b1)

      @pl.when(ii > 0)
      def _():
        wait_out(0, b0)

      compute(0)
      start_out(0, b0)

      wait_in(1, b1)

      @pl.when(ii < npairs - 1)
      def _():
        start_in(0, b0 + 2 * G)

      @pl.when(ii > 0)
      def _():
        wait_out(1, b1)

      compute(1)
      start_out(1, b1)

    wait_out(0, base)
    wait_out(1, base)

  return k(inp_flat, nut, ibase4)


def kernel(input, Nu, uspan):
  nut = Nu.T
  ibase4 = (uspan - P) * 4
  out = _sc_curve_eval(input.reshape(-1), nut, ibase4)
  return out.reshape(B, OUT_DIM, DIM)


# speedup vs baseline: 16.3597x; 16.3597x over previous
import dataclasses
import functools

import jax
import jax.numpy as jnp
from jax import lax
from jax.experimental import pallas as pl
from jax.experimental.pallas import tpu as pltpu
from jax.experimental.pallas import tpu_sc as plsc

B = 1024
M = 1024
P = 2
DIM = 3
OUT_DIM = 2048

NUM_CORES = 2
NUM_SUBCORES = 16
LANES = 16
NUM_TILES = NUM_CORES * NUM_SUBCORES
CURVES_PER_TILE = B // NUM_TILES
NUM_CHUNKS = OUT_DIM // LANES
CPW = M * 4
OBW = OUT_DIM * DIM


def _sc_curve_eval(inp_flat, nut, ibase4):
  mesh = plsc.VectorSubcoreMesh(core_axis_name="c", subcore_axis_name="s")
  cp = pltpu.CompilerParams()
  if "needs_layout_passes" in pltpu.CompilerParams.__dataclass_fields__:
    cp = dataclasses.replace(cp, needs_layout_passes=False)

  G = 4
  ngroups = CURVES_PER_TILE // G
  npairs = ngroups // 2

  @functools.partial(
      pl.kernel,
      compiler_params=cp,
      out_type=jax.ShapeDtypeStruct((B, DIM, OUT_DIM), jnp.float32),
      mesh=mesh,
      scratch_types=(
          [pltpu.VMEM((CPW,), jnp.float32)] * (2 * G)
          + [pltpu.VMEM((DIM, OUT_DIM), jnp.float32)] * (2 * G)
          + [
              pltpu.VMEM((P + 1, OUT_DIM), jnp.float32),
              pltpu.VMEM((OUT_DIM,), jnp.int32),
              pltpu.SemaphoreType.DMA,
              pltpu.SemaphoreType.DMA,
              pltpu.SemaphoreType.DMA,
              pltpu.SemaphoreType.DMA,
              pltpu.SemaphoreType.DMA,
          ]
      ),
  )
  def k(inp_hbm, nut_hbm, idx_hbm, out_hbm, *scratch):
    cpbuf = [scratch[:G], scratch[G:2 * G]]
    obuf = [scratch[2 * G:3 * G], scratch[3 * G:4 * G]]
    nut_v, idx_v, isem0, isem1, osem0, osem1, csem = scratch[4 * G:]
    isem = [isem0, isem1]
    osem = [osem0, osem1]
    wid = lax.axis_index("s") * NUM_CORES + lax.axis_index("c")
    base = wid * CURVES_PER_TILE

    def start_in(buf, b0):
      for g in range(G):
        pltpu.async_copy(inp_hbm.at[b0 + g], cpbuf[buf][g], isem[buf])

    def wait_in(buf, b0):
      for g in range(G):
        pltpu.make_async_copy(inp_hbm.at[b0 + g], cpbuf[buf][g],
                              isem[buf]).wait()

    def start_out(buf, b0):
      for g in range(G):
        pltpu.async_copy(obuf[buf][g], out_hbm.at[b0 + g], osem[buf])

    def wait_out(buf, b0):
      for g in range(G):
        pltpu.make_async_copy(obuf[buf][g], out_hbm.at[b0 + g],
                              osem[buf]).wait()

    pltpu.async_copy(nut_hbm, nut_v, csem)
    pltpu.async_copy(idx_hbm, idx_v, csem)
    start_in(0, base)
    pltpu.make_async_copy(nut_hbm, nut_v, csem).wait()
    pltpu.make_async_copy(idx_hbm, idx_v, csem).wait()

    def compute(buf):
      @pl.loop(0, NUM_CHUNKS)
      def _(kk):
        t0 = kk * LANES
        ibase = idx_v[pl.ds(t0, LANES)]
        nu = [nut_v[j, pl.ds(t0, LANES)] for j in range(3)]
        gidx = [ibase + n for n in range(12)]
        for g in range(G):
          cpg = cpbuf[buf][g]
          acc = [None] * 4
          for c in range(4):
            gs = [plsc.load_gather(cpg, [gidx[4 * j + c]]) for j in range(3)]
            acc[c] = nu[0] * gs[0] + nu[1] * gs[1] + nu[2] * gs[2]
          winv = 1.0 / acc[3]
          og = obuf[buf][g]
          for c in range(3):
            og[c, pl.ds(t0, LANES)] = acc[c] * winv

    @pl.loop(0, npairs)
    def _(ii):
      b0 = base + 2 * G * ii
      b1 = b0 + G
      wait_in(0, b0)
      start_in(1, b1)

      @pl.when(ii > 0)
      def _():
        wait_out(0, b0)

      compute(0)
      start_out(0, b0)

      wait_in(1, b1)

      @pl.when(ii < npairs - 1)
      def _():
        start_in(0, b0 + 2 * G)

      @pl.when(ii > 0)
      def _():
        wait_out(1, b1)

      compute(1)
      start_out(1, b1)

    wait_out(0, base)
    wait_out(1, base)

  return k(inp_flat, nut, ibase4)


def kernel(input, Nu, uspan):
  nut = Nu.T
  ibase4 = (uspan - P) * 4
  out = _sc_curve_eval(input.reshape(B, M * 4), nut, ibase4)
  return out.swapaxes(1, 2)


# speedup vs baseline: 23.6564x; 1.4460x over previous
import dataclasses
import functools

import jax
import jax.numpy as jnp
from jax import lax
from jax.experimental import pallas as pl
from jax.experimental.pallas import tpu as pltpu
from jax.experimental.pallas import tpu_sc as plsc

B = 1024
M = 1024
P = 2
DIM = 3
OUT_DIM = 2048

NUM_CORES = 2
NUM_SUBCORES = 16
LANES = 16
NUM_TILES = NUM_CORES * NUM_SUBCORES
CURVES_PER_TILE = B // NUM_TILES
NUM_CHUNKS = OUT_DIM // LANES
CPW = M * 4
OBW = OUT_DIM * DIM


def _sc_curve_eval(inp_pl, nut, rbase):
  mesh = plsc.VectorSubcoreMesh(core_axis_name="c", subcore_axis_name="s")
  cp = pltpu.CompilerParams()
  if "needs_layout_passes" in pltpu.CompilerParams.__dataclass_fields__:
    cp = dataclasses.replace(cp, needs_layout_passes=False)

  G = 4
  ngroups = CURVES_PER_TILE // G
  npairs = ngroups // 2

  @functools.partial(
      pl.kernel,
      compiler_params=cp,
      out_type=jax.ShapeDtypeStruct((B, DIM, OUT_DIM), jnp.float32),
      mesh=mesh,
      scratch_types=(
          [pltpu.VMEM((4, M), jnp.float32)] * (2 * G)
          + [pltpu.VMEM((DIM, OUT_DIM), jnp.float32)] * (2 * G)
          + [
              pltpu.VMEM((P + 1, OUT_DIM), jnp.float32),
              pltpu.VMEM((OUT_DIM,), jnp.int32),
              pltpu.SemaphoreType.DMA,
              pltpu.SemaphoreType.DMA,
              pltpu.SemaphoreType.DMA,
              pltpu.SemaphoreType.DMA,
              pltpu.SemaphoreType.DMA,
          ]
      ),
  )
  def k(inp_hbm, nut_hbm, idx_hbm, out_hbm, *scratch):
    cpbuf = [scratch[:G], scratch[G:2 * G]]
    obuf = [scratch[2 * G:3 * G], scratch[3 * G:4 * G]]
    nut_v, idx_v, isem0, isem1, osem0, osem1, csem = scratch[4 * G:]
    isem = [isem0, isem1]
    osem = [osem0, osem1]
    wid = lax.axis_index("s") * NUM_CORES + lax.axis_index("c")
    base = wid * CURVES_PER_TILE

    def start_in(buf, b0):
      for g in range(G):
        pltpu.async_copy(inp_hbm.at[b0 + g], cpbuf[buf][g], isem[buf])

    def wait_in(buf, b0):
      for g in range(G):
        pltpu.make_async_copy(inp_hbm.at[b0 + g], cpbuf[buf][g],
                              isem[buf]).wait()

    def start_out(buf, b0):
      for g in range(G):
        pltpu.async_copy(obuf[buf][g], out_hbm.at[b0 + g], osem[buf])

    def wait_out(buf, b0):
      for g in range(G):
        pltpu.make_async_copy(obuf[buf][g], out_hbm.at[b0 + g],
                              osem[buf]).wait()

    pltpu.async_copy(nut_hbm, nut_v, csem)
    pltpu.async_copy(idx_hbm, idx_v, csem)
    start_in(0, base)
    pltpu.make_async_copy(nut_hbm, nut_v, csem).wait()
    pltpu.make_async_copy(idx_hbm, idx_v, csem).wait()

    csplat = [jnp.full((LANES,), c, jnp.int32) for c in range(4)]

    def compute(buf):
      @pl.loop(0, NUM_CHUNKS)
      def _(kk):
        t0 = kk * LANES
        r0 = idx_v[pl.ds(t0, LANES)]
        rows = [r0, r0 + 1, r0 + 2]
        nu = [nut_v[j, pl.ds(t0, LANES)] for j in range(3)]
        for g in range(G):
          cpg = cpbuf[buf][g]
          acc = [None] * 4
          for c in range(4):
            gs = [plsc.load_gather(cpg, [csplat[c], rows[j]])
                  for j in range(3)]
            acc[c] = nu[0] * gs[0] + nu[1] * gs[1] + nu[2] * gs[2]
          winv = 1.0 / acc[3]
          og = obuf[buf][g]
          for c in range(3):
            og[c, pl.ds(t0, LANES)] = acc[c] * winv

    @pl.loop(0, npairs)
    def _(ii):
      b0 = base + 2 * G * ii
      b1 = b0 + G
      wait_in(0, b0)
      start_in(1, b1)

      @pl.when(ii > 0)
      def _():
        wait_out(0, b0)

      compute(0)
      start_out(0, b0)

      wait_in(1, b1)

      @pl.when(ii < npairs - 1)
      def _():
        start_in(0, b0 + 2 * G)

      @pl.when(ii > 0)
      def _():
        wait_out(1, b1)

      compute(1)
      start_out(1, b1)

    wait_out(0, base)
    wait_out(1, base)

  return k(inp_pl, nut, rbase)


def kernel(input, Nu, uspan):
  nut = Nu.T
  rbase = uspan - P
  out = _sc_curve_eval(input.swapaxes(1, 2), nut, rbase)
  return out.swapaxes(1, 2)


# speedup vs baseline: 44.5728x; 1.8842x over previous
import dataclasses
import functools

import jax
import jax.numpy as jnp
from jax import lax
from jax.experimental import pallas as pl
from jax.experimental.pallas import tpu as pltpu
from jax.experimental.pallas import tpu_sc as plsc

B = 1024
M = 1024
P = 2
DIM = 3
OUT_DIM = 2048

NUM_CORES = 2
NUM_SUBCORES = 16
LANES = 16
NUM_TILES = NUM_CORES * NUM_SUBCORES
CURVES_PER_TILE = B // NUM_TILES
NUM_CHUNKS = OUT_DIM // LANES
CPW = M * 4
OBW = OUT_DIM * DIM


def _sc_curve_eval(inp_pl, nut, rbase):
  mesh = plsc.VectorSubcoreMesh(core_axis_name="c", subcore_axis_name="s")
  cp = pltpu.CompilerParams()
  if "needs_layout_passes" in pltpu.CompilerParams.__dataclass_fields__:
    cp = dataclasses.replace(cp, needs_layout_passes=False)

  G = 4
  ngroups = CURVES_PER_TILE // G
  npairs = ngroups // 2

  @functools.partial(
      pl.kernel,
      compiler_params=cp,
      out_type=jax.ShapeDtypeStruct((B, DIM, OUT_DIM), jnp.float32),
      mesh=mesh,
      scratch_types=(
          [pltpu.VMEM((4, M), jnp.float32)] * (2 * G)
          + [pltpu.VMEM((DIM, OUT_DIM), jnp.float32)] * (2 * G)
          + [
              pltpu.VMEM((P + 1, OUT_DIM), jnp.float32),
              pltpu.VMEM((OUT_DIM,), jnp.int32),
              pltpu.SemaphoreType.DMA,
              pltpu.SemaphoreType.DMA,
              pltpu.SemaphoreType.DMA,
              pltpu.SemaphoreType.DMA,
              pltpu.SemaphoreType.DMA,
          ]
      ),
  )
  def k(inp_hbm, nut_hbm, idx_hbm, out_hbm, *scratch):
    cpbuf = [scratch[:G], scratch[G:2 * G]]
    obuf = [scratch[2 * G:3 * G], scratch[3 * G:4 * G]]
    nut_v, idx_v, isem0, isem1, osem0, osem1, csem = scratch[4 * G:]
    isem = [isem0, isem1]
    osem = [osem0, osem1]
    wid = lax.axis_index("s") * NUM_CORES + lax.axis_index("c")
    base = wid * CURVES_PER_TILE

    def start_in(buf, b0):
      for g in range(G):
        pltpu.async_copy(inp_hbm.at[b0 + g], cpbuf[buf][g], isem[buf])

    def wait_in(buf, b0):
      for g in range(G):
        pltpu.make_async_copy(inp_hbm.at[b0 + g], cpbuf[buf][g],
                              isem[buf]).wait()

    def start_out(buf, b0):
      for g in range(G):
        pltpu.async_copy(obuf[buf][g], out_hbm.at[b0 + g], osem[buf])

    def wait_out(buf, b0):
      for g in range(G):
        pltpu.make_async_copy(obuf[buf][g], out_hbm.at[b0 + g],
                              osem[buf]).wait()

    pltpu.async_copy(nut_hbm, nut_v, csem)
    pltpu.async_copy(idx_hbm, idx_v, csem)
    start_in(0, base)
    pltpu.make_async_copy(nut_hbm, nut_v, csem).wait()
    pltpu.make_async_copy(idx_hbm, idx_v, csem).wait()

    csplat = [jnp.full((LANES,), c, jnp.int32) for c in range(4)]

    def compute(buf):
      @functools.partial(plsc.parallel_loop, 0, NUM_CHUNKS, unroll=1)
      def _(kk):
        t0 = kk * LANES
        r0 = idx_v[pl.ds(t0, LANES)]
        rows = [r0, r0 + 1, r0 + 2]
        nu = [nut_v[j, pl.ds(t0, LANES)] for j in range(3)]
        for g in range(G):
          cpg = cpbuf[buf][g]
          acc = [None] * 4
          for c in range(4):
            gs = [plsc.load_gather(cpg, [csplat[c], rows[j]])
                  for j in range(3)]
            acc[c] = nu[0] * gs[0] + nu[1] * gs[1] + nu[2] * gs[2]
          winv = 1.0 / acc[3]
          og = obuf[buf][g]
          for c in range(3):
            og[c, pl.ds(t0, LANES)] = acc[c] * winv

    @pl.loop(0, npairs)
    def _(ii):
      b0 = base + 2 * G * ii
      b1 = b0 + G
      wait_in(0, b0)
      start_in(1, b1)

      @pl.when(ii > 0)
      def _():
        wait_out(0, b0)

      compute(0)
      start_out(0, b0)

      wait_in(1, b1)

      @pl.when(ii < npairs - 1)
      def _():
        start_in(0, b0 + 2 * G)

      @pl.when(ii > 0)
      def _():
        wait_out(1, b1)

      compute(1)
      start_out(1, b1)

    wait_out(0, base)
    wait_out(1, base)

  return k(inp_pl, nut, rbase)


def kernel(input, Nu, uspan):
  nut = Nu.T
  rbase = uspan - P
  out = _sc_curve_eval(input.swapaxes(1, 2), nut, rbase)
  return out.swapaxes(1, 2)


# speedup vs baseline: 45.3807x; 1.0181x over previous
import dataclasses
import functools

import jax
import jax.numpy as jnp
from jax import lax
from jax.experimental import pallas as pl
from jax.experimental.pallas import tpu as pltpu
from jax.experimental.pallas import tpu_sc as plsc

B = 1024
M = 1024
P = 2
DIM = 3
OUT_DIM = 2048

NUM_CORES = 2
NUM_SUBCORES = 16
LANES = 16
NUM_TILES = NUM_CORES * NUM_SUBCORES
CURVES_PER_TILE = B // NUM_TILES
NUM_CHUNKS = OUT_DIM // LANES
CPW = M * 4
OBW = OUT_DIM * DIM


def _sc_curve_eval(inp_pl, nut, rbase):
  mesh = plsc.VectorSubcoreMesh(core_axis_name="c", subcore_axis_name="s")
  cp = pltpu.CompilerParams()
  if "needs_layout_passes" in pltpu.CompilerParams.__dataclass_fields__:
    cp = dataclasses.replace(cp, needs_layout_passes=False)

  G = 4
  ngroups = CURVES_PER_TILE // G
  npairs = ngroups // 2

  @functools.partial(
      pl.kernel,
      compiler_params=cp,
      out_type=jax.ShapeDtypeStruct((B, DIM, OUT_DIM), jnp.float32),
      mesh=mesh,
      scratch_types=(
          [pltpu.VMEM((4, M), jnp.float32)] * (2 * G)
          + [pltpu.VMEM((DIM, OUT_DIM), jnp.float32)] * (2 * G)
          + [
              pltpu.VMEM((P + 1, OUT_DIM), jnp.float32),
              pltpu.VMEM((OUT_DIM,), jnp.int32),
              pltpu.SemaphoreType.DMA,
              pltpu.SemaphoreType.DMA,
              pltpu.SemaphoreType.DMA,
              pltpu.SemaphoreType.DMA,
              pltpu.SemaphoreType.DMA,
          ]
      ),
  )
  def k(inp_hbm, nut_hbm, idx_hbm, out_hbm, *scratch):
    cpbuf = [scratch[:G], scratch[G:2 * G]]
    obuf = [scratch[2 * G:3 * G], scratch[3 * G:4 * G]]
    nut_v, idx_v, isem0, isem1, osem0, osem1, csem = scratch[4 * G:]
    isem = [isem0, isem1]
    osem = [osem0, osem1]
    wid = lax.axis_index("s") * NUM_CORES + lax.axis_index("c")
    base = wid * CURVES_PER_TILE

    def start_in(buf, b0):
      for g in range(G):
        pltpu.async_copy(inp_hbm.at[b0 + g], cpbuf[buf][g], isem[buf])

    def wait_in(buf, b0):
      for g in range(G):
        pltpu.make_async_copy(inp_hbm.at[b0 + g], cpbuf[buf][g],
                              isem[buf]).wait()

    def start_out(buf, b0):
      for g in range(G):
        pltpu.async_copy(obuf[buf][g], out_hbm.at[b0 + g], osem[buf])

    def wait_out(buf, b0):
      for g in range(G):
        pltpu.make_async_copy(obuf[buf][g], out_hbm.at[b0 + g],
                              osem[buf]).wait()

    pltpu.async_copy(nut_hbm, nut_v, csem)
    pltpu.async_copy(idx_hbm, idx_v, csem)
    start_in(0, base)
    pltpu.make_async_copy(nut_hbm, nut_v, csem).wait()
    pltpu.make_async_copy(idx_hbm, idx_v, csem).wait()

    csplat = [jnp.full((LANES,), c, jnp.int32) for c in range(4)]

    def compute(buf):
      plsc.subcore_barrier()

      @functools.partial(plsc.parallel_loop, 0, NUM_CHUNKS, unroll=1)
      def _(kk):
        t0 = kk * LANES
        r0 = idx_v[pl.ds(t0, LANES)]
        rows = [r0, r0 + 1, r0 + 2]
        nu = [nut_v[j, pl.ds(t0, LANES)] for j in range(3)]
        for g in range(G):
          cpg = cpbuf[buf][g]
          acc = [None] * 4
          for c in range(4):
            gs = [plsc.load_gather(cpg, [csplat[c], rows[j]])
                  for j in range(3)]
            acc[c] = nu[0] * gs[0] + nu[1] * gs[1] + nu[2] * gs[2]
          winv = 1.0 / acc[3]
          og = obuf[buf][g]
          for c in range(3):
            og[c, pl.ds(t0, LANES)] = acc[c] * winv

      plsc.subcore_barrier()

    @pl.loop(0, npairs)
    def _(ii):
      b0 = base + 2 * G * ii
      b1 = b0 + G
      wait_in(0, b0)
      start_in(1, b1)

      @pl.when(ii > 0)
      def _():
        wait_out(0, b0)

      compute(0)
      start_out(0, b0)

      wait_in(1, b1)

      @pl.when(ii < npairs - 1)
      def _():
        start_in(0, b0 + 2 * G)

      @pl.when(ii > 0)
      def _():
        wait_out(1, b1)

      compute(1)
      start_out(1, b1)

    wait_out(0, base)
    wait_out(1, base)

  return k(inp_pl, nut, rbase)


def kernel(input, Nu, uspan):
  nut = Nu.T
  rbase = uspan - P
  out = _sc_curve_eval(input.swapaxes(1, 2), nut, rbase)
  return out.swapaxes(1, 2)
